# baseline (speedup 1.0000x reference)
import functools
import math

import jax
import jax.numpy as jnp
from jax import lax
from jax.experimental import pallas as pl
from jax.experimental.pallas import tpu as pltpu
from jax.experimental.pallas import tpu_sc as plsc

NC = 2
NS = 16
NW = NC * NS
LANES = 16

D = 64
NTOK = 1000000
BATCH = 4096
HIST = 200
NB = BATCH // 128
ND = D // 8
SCALE = math.sqrt(float(D))

_mesh = plsc.VectorSubcoreMesh(
    core_axis_name="c", subcore_axis_name="s", num_cores=NC, num_subcores=NS
)


@functools.partial(
    pl.kernel,
    out_type=jax.ShapeDtypeStruct((HIST, ND, NB, 8, 128), jnp.float32),
    mesh=_mesh,
    scratch_types=[
        pltpu.VMEM((128,), jnp.int32),
        pltpu.VMEM((128,), jnp.int32),
        pltpu.VMEM((128, D), jnp.float32),
        pltpu.VMEM((128, D), jnp.float32),
        pltpu.VMEM((ND, 8, 129), jnp.float32),
        pltpu.VMEM((ND, 8, 129), jnp.float32),
        pltpu.SemaphoreType.DMA,
        pltpu.SemaphoreType.DMA,
        pltpu.SemaphoreType.DMA,
        pltpu.SemaphoreType.DMA,
    ],
    compiler_params=pltpu.CompilerParams(
        use_tc_tiling_on_sc=False, needs_layout_passes=False
    ),
)
def _emb_lookup(
    table_hbm, srct_hbm, out_hbm,
    idx_a, idx_b, rows_a, rows_b, tiles_a, tiles_b, sem_a, sem_b, wsem_a, wsem_b,
):
    wid = lax.axis_index("s") * NC + lax.axis_index("c")
    col0 = wid * 128

    def stage(h, idx_v, rows_v, sem):
        pltpu.sync_copy(srct_hbm.at[h, pl.ds(col0, 128)], idx_v)
        for o in range(0, 128, LANES):
            sl = pl.ds(o, LANES)
            idx_v[sl] = idx_v[sl] * 2
        pltpu.async_copy(table_hbm.at[idx_v], rows_v, sem)

    def gather_wait(idx_v, rows_v, sem):
        pltpu.make_async_copy(table_hbm.at[idx_v], rows_v, sem).wait()

    zeros16 = jnp.zeros((LANES,), jnp.int32)
    iota16 = lax.iota(jnp.int32, LANES)
    tdvs = [lax.shift_right_logical(iota16 + k * LANES, 3) for k in range(4)]
    svs = [lax.bitwise_and(iota16 + k * LANES, 7) for k in range(4)]

    def consume(h, rows_v, tiles_v, wsem):
        @plsc.parallel_loop(0, 128, unroll=4)
        def _t(l):
            lv = zeros16 + l
            for k in range(4):
                v = rows_v[l, pl.ds(k * LANES, LANES)]
                plsc.store_scatter(tiles_v, [tdvs[k], svs[k], lv], v)

        pltpu.async_copy(
            tiles_v.at[:, :, pl.ds(0, 128)], out_hbm.at[h, :, wid], wsem
        )

    def write_wait(h, tiles_v, wsem):
        pltpu.make_async_copy(
            tiles_v.at[:, :, pl.ds(0, 128)], out_hbm.at[h, :, wid], wsem
        ).wait()

    stage(0, idx_a, rows_a, sem_a)

    @pl.loop(0, HIST, step=2)
    def _step(h):
        stage(h + 1, idx_b, rows_b, sem_b)
        gather_wait(idx_a, rows_a, sem_a)

        @pl.when(h >= 2)
        def _wa():
            write_wait(h - 2, tiles_a, wsem_a)

        consume(h, rows_a, tiles_a, wsem_a)

        @pl.when(h + 2 < HIST)
        def _prefetch():
            stage(h + 2, idx_a, rows_a, sem_a)

        gather_wait(idx_b, rows_b, sem_b)

        @pl.when(h >= 2)
        def _wb():
            write_wait(h - 1, tiles_b, wsem_b)

        consume(h + 1, rows_b, tiles_b, wsem_b)

    write_wait(HIST - 2, tiles_a, wsem_a)
    write_wait(HIST - 1, tiles_b, wsem_b)


_TBLK = 8192


def _fmt_body(in_ref, out_ref):
    x = in_ref[...]
    eye = jnp.eye(D, dtype=jnp.float32) * SCALE
    y = lax.dot_general(
        x, eye, (((0,), (0,)), ((), ())), preferred_element_type=jnp.float32
    )
    out_ref[...] = jnp.pad(y, ((0, 0), (0, D)))


_tc_format = pl.pallas_call(
    _fmt_body,
    grid=((NTOK + _TBLK - 1) // _TBLK,),
    in_specs=[pl.BlockSpec((D, _TBLK), lambda i: (0, i))],
    out_specs=pl.BlockSpec((_TBLK, 2 * D), lambda i: (i, 0)),
    out_shape=jax.ShapeDtypeStruct((NTOK, 2 * D), jnp.float32),
)


def kernel(src, emb_weight):
    src_t = src.T.astype(jnp.int32)
    table3 = _tc_format(emb_weight.T).reshape(2 * NTOK, D)
    x = _emb_lookup(table3, src_t)
    out = jnp.transpose(x, (2, 4, 0, 1, 3))
    return out.reshape(BATCH, HIST, D)


# speedup vs baseline: 1.0115x; 1.0115x over previous
import functools
import math

import jax
import jax.numpy as jnp
from jax import lax
from jax.experimental import pallas as pl
from jax.experimental.pallas import tpu as pltpu
from jax.experimental.pallas import tpu_sc as plsc

NC = 2
NS = 16
NW = NC * NS
LANES = 16

D = 64
NTOK = 1000000
BATCH = 4096
HIST = 200
NB = BATCH // 128
ND = D // 8
SCALE = math.sqrt(float(D))

_mesh = plsc.VectorSubcoreMesh(
    core_axis_name="c", subcore_axis_name="s", num_cores=NC, num_subcores=NS
)


@functools.partial(
    pl.kernel,
    out_type=jax.ShapeDtypeStruct((HIST, ND, NB, 8, 128), jnp.float32),
    mesh=_mesh,
    scratch_types=[
        pltpu.VMEM((128,), jnp.int32),
        pltpu.VMEM((128,), jnp.int32),
        pltpu.VMEM((128, D), jnp.float32),
        pltpu.VMEM((128, D), jnp.float32),
        pltpu.VMEM((ND, 8, 129), jnp.float32),
        pltpu.VMEM((ND, 8, 129), jnp.float32),
        pltpu.SemaphoreType.DMA,
        pltpu.SemaphoreType.DMA,
        pltpu.SemaphoreType.DMA,
        pltpu.SemaphoreType.DMA,
    ],
    compiler_params=pltpu.CompilerParams(
        use_tc_tiling_on_sc=False, needs_layout_passes=False
    ),
)
def _emb_lookup(
    table_hbm, srct_hbm, out_hbm,
    idx_a, idx_b, rows_a, rows_b, tiles_a, tiles_b, sem_a, sem_b, wsem_a, wsem_b,
):
    wid = lax.axis_index("s") * NC + lax.axis_index("c")
    col0 = wid * 128

    def stage(h, idx_v, rows_v, sem):
        pltpu.sync_copy(srct_hbm.at[h, pl.ds(col0, 128)], idx_v)
        for o in range(0, 128, LANES):
            sl = pl.ds(o, LANES)
            idx_v[sl] = idx_v[sl] * 2
        pltpu.async_copy(table_hbm.at[idx_v], rows_v, sem)

    def gather_wait(idx_v, rows_v, sem):
        pltpu.make_async_copy(table_hbm.at[idx_v], rows_v, sem).wait()

    zeros16 = jnp.zeros((LANES,), jnp.int32)
    iota16 = lax.iota(jnp.int32, LANES)
    tdvs = [lax.shift_right_logical(iota16 + k * LANES, 3) for k in range(4)]
    svs = [lax.bitwise_and(iota16 + k * LANES, 7) for k in range(4)]

    def consume(h, rows_v, tiles_v, wsem):
        @plsc.parallel_loop(0, 128, unroll=4)
        def _t(l):
            lv = zeros16 + l
            for k in range(4):
                v = rows_v[l, pl.ds(k * LANES, LANES)]
                plsc.store_scatter(tiles_v, [tdvs[k], svs[k], lv], v)

        pltpu.async_copy(
            tiles_v.at[:, :, pl.ds(0, 128)], out_hbm.at[h, :, wid], wsem
        )

    def write_wait(h, tiles_v, wsem):
        pltpu.make_async_copy(
            tiles_v.at[:, :, pl.ds(0, 128)], out_hbm.at[h, :, wid], wsem
        ).wait()

    stage(0, idx_a, rows_a, sem_a)

    @pl.loop(0, HIST, step=2)
    def _step(h):
        stage(h + 1, idx_b, rows_b, sem_b)
        gather_wait(idx_a, rows_a, sem_a)

        @pl.when(h >= 2)
        def _wa():
            write_wait(h - 2, tiles_a, wsem_a)

        consume(h, rows_a, tiles_a, wsem_a)

        @pl.when(h + 2 < HIST)
        def _prefetch():
            stage(h + 2, idx_a, rows_a, sem_a)

        gather_wait(idx_b, rows_b, sem_b)

        @pl.when(h >= 2)
        def _wb():
            write_wait(h - 1, tiles_b, wsem_b)

        consume(h + 1, rows_b, tiles_b, wsem_b)

    write_wait(HIST - 2, tiles_a, wsem_a)
    write_wait(HIST - 1, tiles_b, wsem_b)


_TBLK = 8192


def _fmt_body(in_ref, out_ref):
    x = in_ref[...]
    y = jnp.transpose(x) * SCALE
    out_ref[...] = jnp.pad(y, ((0, 0), (0, D)))


_tc_format = pl.pallas_call(
    _fmt_body,
    grid=((NTOK + _TBLK - 1) // _TBLK,),
    in_specs=[pl.BlockSpec((D, _TBLK), lambda i: (0, i))],
    out_specs=pl.BlockSpec((_TBLK, 2 * D), lambda i: (i, 0)),
    out_shape=jax.ShapeDtypeStruct((NTOK, 2 * D), jnp.float32),
)


def kernel(src, emb_weight):
    src_t = src.T.astype(jnp.int32)
    table3 = _tc_format(emb_weight.T).reshape(2 * NTOK, D)
    x = _emb_lookup(table3, src_t)
    out = jnp.transpose(x, (2, 4, 0, 1, 3))
    return out.reshape(BATCH, HIST, D)


# speedup vs baseline: 1.0133x; 1.0018x over previous
import functools
import math

import jax
import jax.numpy as jnp
from jax import lax
from jax.experimental import pallas as pl
from jax.experimental.pallas import tpu as pltpu
from jax.experimental.pallas import tpu_sc as plsc

NC = 2
NS = 16
NW = NC * NS
LANES = 16

D = 64
NTOK = 1000000
BATCH = 4096
HIST = 200
NB = BATCH // 128
ND = D // 8
SCALE = math.sqrt(float(D))

_mesh = plsc.VectorSubcoreMesh(
    core_axis_name="c", subcore_axis_name="s", num_cores=NC, num_subcores=NS
)


@functools.partial(
    pl.kernel,
    out_type=jax.ShapeDtypeStruct((HIST, ND, NB, 8, 128), jnp.float32),
    mesh=_mesh,
    scratch_types=[
        pltpu.VMEM((128,), jnp.int32),
        pltpu.VMEM((128,), jnp.int32),
        pltpu.VMEM((128, D), jnp.float32),
        pltpu.VMEM((128, D), jnp.float32),
        pltpu.VMEM((ND, 8, 129), jnp.float32),
        pltpu.VMEM((ND, 8, 129), jnp.float32),
        pltpu.SemaphoreType.DMA,
        pltpu.SemaphoreType.DMA,
        pltpu.SemaphoreType.DMA,
        pltpu.SemaphoreType.DMA,
    ],
    compiler_params=pltpu.CompilerParams(
        use_tc_tiling_on_sc=False, needs_layout_passes=False
    ),
)
def _emb_lookup(
    table_hbm, srct_hbm, out_hbm,
    idx_a, idx_b, rows_a, rows_b, tiles_a, tiles_b, sem_a, sem_b, wsem_a, wsem_b,
):
    wid = lax.axis_index("s") * NC + lax.axis_index("c")
    col0 = wid * 128

    def stage(h, idx_v, rows_v, sem):
        pltpu.sync_copy(srct_hbm.at[h, pl.ds(col0, 128)], idx_v)
        for o in range(0, 128, LANES):
            sl = pl.ds(o, LANES)
            idx_v[sl] = idx_v[sl] * 2
        pltpu.async_copy(table_hbm.at[idx_v], rows_v, sem)

    def gather_wait(idx_v, rows_v, sem):
        pltpu.make_async_copy(table_hbm.at[idx_v], rows_v, sem).wait()

    zeros16 = jnp.zeros((LANES,), jnp.int32)
    iota16 = lax.iota(jnp.int32, LANES)
    tdvs = [lax.shift_right_logical(iota16 + k * LANES, 3) for k in range(4)]
    svs = [lax.bitwise_and(iota16 + k * LANES, 7) for k in range(4)]

    def consume(h, rows_v, tiles_v, wsem):
        @plsc.parallel_loop(0, 128, unroll=8)
        def _t(l):
            lv = zeros16 + l
            for k in range(4):
                v = rows_v[l, pl.ds(k * LANES, LANES)]
                plsc.store_scatter(tiles_v, [tdvs[k], svs[k], lv], v)

        pltpu.async_copy(
            tiles_v.at[:, :, pl.ds(0, 128)], out_hbm.at[h, :, wid], wsem
        )

    def write_wait(h, tiles_v, wsem):
        pltpu.make_async_copy(
            tiles_v.at[:, :, pl.ds(0, 128)], out_hbm.at[h, :, wid], wsem
        ).wait()

    stage(0, idx_a, rows_a, sem_a)

    @pl.loop(0, HIST, step=2)
    def _step(h):
        stage(h + 1, idx_b, rows_b, sem_b)
        gather_wait(idx_a, rows_a, sem_a)

        @pl.when(h >= 2)
        def _wa():
            write_wait(h - 2, tiles_a, wsem_a)

        consume(h, rows_a, tiles_a, wsem_a)

        @pl.when(h + 2 < HIST)
        def _prefetch():
            stage(h + 2, idx_a, rows_a, sem_a)

        gather_wait(idx_b, rows_b, sem_b)

        @pl.when(h >= 2)
        def _wb():
            write_wait(h - 1, tiles_b, wsem_b)

        consume(h + 1, rows_b, tiles_b, wsem_b)

    write_wait(HIST - 2, tiles_a, wsem_a)
    write_wait(HIST - 1, tiles_b, wsem_b)


_TBLK = 8192


def _fmt_body(in_ref, out_ref):
    x = in_ref[...]
    y = jnp.transpose(x) * SCALE
    out_ref[...] = jnp.pad(y, ((0, 0), (0, D)))


_tc_format = pl.pallas_call(
    _fmt_body,
    grid=((NTOK + _TBLK - 1) // _TBLK,),
    in_specs=[pl.BlockSpec((D, _TBLK), lambda i: (0, i))],
    out_specs=pl.BlockSpec((_TBLK, 2 * D), lambda i: (i, 0)),
    out_shape=jax.ShapeDtypeStruct((NTOK, 2 * D), jnp.float32),
)


def kernel(src, emb_weight):
    src_t = src.T.astype(jnp.int32)
    table3 = _tc_format(emb_weight.T).reshape(2 * NTOK, D)
    x = _emb_lookup(table3, src_t)
    out = jnp.transpose(x, (2, 4, 0, 1, 3))
    return out.reshape(BATCH, HIST, D)
